# baseline (speedup 1.0000x reference)
import jax
import jax.numpy as jnp
from jax.experimental import pallas as pl
from jax.experimental.pallas import tpu as pltpu

DIM = 96
E = 8
H = 224
W = 224
WP = 256
HP = 232
TB = 2048
OB = 8 * W
GRID = HP * WP // TB
HALO = 3 * WP + 3
EPS = 1e-06


def _moecn_kernel(xp_ref, xc_ref, xn_ref, xres_ref, k_ref, cb_ref, lng_ref,
                  lnb_ref, gw_ref, w1s_ref, w2s_ref, ls_ref, out_ref,
                  xln_s, yp_s):
    xln = xln_s[...]
    logits = jnp.dot(gw_ref[...], xln, preferred_element_type=jnp.float32)
    mx = jnp.max(logits, axis=0, keepdims=True)
    taken = jnp.zeros((1, TB), jnp.bool_)
    masks = []
    for e in range(E):
        hit = (logits[e:e + 1, :] == mx) & (~taken)
        taken = taken | hit
        masks.append(hit.astype(jnp.bfloat16))
    zrow = jnp.zeros((E, TB), jnp.bfloat16)

    xb = xln.astype(jnp.bfloat16)
    xs = jnp.concatenate([masks[e] * xb for e in range(E)]
                         + masks + [zrow], axis=0)
    hsel = jnp.dot(w1s_ref[...], xs, preferred_element_type=jnp.float32)
    g = 0.5 * hsel * (1.0 + jax.lax.erf(hsel * 0.7071067811865476))
    gb = g.astype(jnp.bfloat16)
    gs = jnp.concatenate([masks[e] * gb for e in range(E)]
                         + masks + [zrow], axis=0)
    y = jnp.dot(w2s_ref[...], gs, preferred_element_type=jnp.float32)

    pieces = []
    for r in range(8):
        if r <= 4:
            pieces.append(yp_s[:, WP * (r + 3) + 3:WP * (r + 3) + 3 + W])
        else:
            pieces.append(y[:, WP * (r - 5) + 3:WP * (r - 5) + 3 + W])
    ycat = jnp.concatenate(pieces, axis=1)
    out_ref[...] = xres_ref[...] + ls_ref[...] * ycat
    yp_s[...] = y

    lo = TB - HALO
    parts = []
    for dw in range(7):
        xw = jnp.concatenate(
            [xp_ref[:, lo + dw:], xc_ref[...], xn_ref[:, :HALO - 6 + dw]],
            axis=1)
        p = None
        for dh in range(7):
            tap = k_ref[:, dh * 7 + dw:dh * 7 + dw + 1]
            t = xw[:, dh * WP:dh * WP + TB] * tap
            p = t if p is None else p + t
        parts.append(p)
    acc = ((parts[0] + parts[1]) + (parts[2] + parts[3])) \
        + ((parts[4] + parts[5]) + parts[6])
    xc = acc.astype(jnp.float32) + cb_ref[...]

    mu = jnp.mean(xc, axis=0, keepdims=True)
    var = jnp.mean(xc * xc, axis=0, keepdims=True) - mu * mu
    xln_s[...] = ((xc - mu) * jax.lax.rsqrt(var + EPS) * lng_ref[...]
                  + lnb_ref[...])


def kernel(input, conv_w, conv_b, ln_g, ln_b, gate_w, w1, b1, w2, b2, ls):
    x3 = jnp.pad(input[0], ((0, 0), (3, HP - H - 3), (3, WP - W - 3)))
    x2 = x3.reshape(DIM, HP * WP).astype(jnp.bfloat16)
    xres = input.reshape(DIM, H * W)

    k = conv_w[:, 0, :, :].reshape(DIM, 49)
    k = jnp.pad(k, ((0, 0), (0, 7))).astype(jnp.bfloat16)
    cb = conv_b.reshape(DIM, 1)
    lng = ln_g.reshape(DIM, 1)
    lnb = ln_b.reshape(DIM, 1)
    lsr = ls.reshape(DIM, 1)
    zcol = jnp.zeros((DIM, E), jnp.float32)
    w1s = jnp.concatenate(
        [jnp.transpose(w1, (1, 0, 2)).reshape(DIM, E * DIM), b1.T, zcol],
        axis=1).astype(jnp.bfloat16)
    w2s = jnp.concatenate(
        [jnp.transpose(w2, (1, 0, 2)).reshape(DIM, E * DIM), b2.T, zcol],
        axis=1).astype(jnp.bfloat16)

    blk = lambda f: pl.BlockSpec((DIM, TB), lambda i: (0, f(i)))
    full = lambda a: pl.BlockSpec(a.shape, lambda i: (0,) * a.ndim)

    y = pl.pallas_call(
        _moecn_kernel,
        grid=(GRID + 1,),
        in_specs=[
            blk(lambda i: jnp.clip(i - 1, 0, GRID - 1)),
            blk(lambda i: jnp.minimum(i, GRID - 1)),
            blk(lambda i: jnp.minimum(i + 1, GRID - 1)),
            pl.BlockSpec((DIM, OB),
                         lambda i: (0, jnp.clip(i - 2, 0, H // 8 - 1))),
            full(k), full(cb), full(lng), full(lnb),
            full(gate_w), full(w1s), full(w2s), full(lsr),
        ],
        out_specs=pl.BlockSpec((DIM, OB),
                               lambda i: (0, jnp.clip(i - 2, 0, H // 8 - 1))),
        out_shape=jax.ShapeDtypeStruct((DIM, H * W), jnp.float32),
        scratch_shapes=[pltpu.VMEM((DIM, TB), jnp.float32),
                        pltpu.VMEM((DIM, TB), jnp.float32)],
    )(x2, x2, x2, xres, k, cb, lng, lnb, gate_w, w1s, w2s, lsr)

    return y.reshape(1, DIM, H, W)


# speedup vs baseline: 1.0804x; 1.0804x over previous
import jax
import jax.numpy as jnp
from jax.experimental import pallas as pl
from jax.experimental.pallas import tpu as pltpu

DIM = 96
E = 8
H = 224
W = 224
WP = 256
HP = 232
TB = 2048
GRID = HP * WP // TB
HALO = 3 * WP + 3
EPS = 1e-06


def _moecn_kernel(xp_ref, xc_ref, xn_ref, k_ref, cb_ref, lng_ref, lnb_ref,
                  gw_ref, w1s_ref, w2s_ref, out_ref, xln_s):
    xln = xln_s[...]
    logits = jnp.dot(gw_ref[...], xln, preferred_element_type=jnp.float32)
    mx = jnp.max(logits, axis=0, keepdims=True)
    taken = jnp.zeros((1, TB), jnp.bool_)
    masks = []
    for e in range(E):
        hit = (logits[e:e + 1, :] == mx) & (~taken)
        taken = taken | hit
        masks.append(hit.astype(jnp.bfloat16))
    zrow = jnp.zeros((E, TB), jnp.bfloat16)

    xb = xln.astype(jnp.bfloat16)
    xs = jnp.concatenate([masks[e] * xb for e in range(E)]
                         + masks + [zrow], axis=0)
    hsel = jnp.dot(w1s_ref[...], xs, preferred_element_type=jnp.float32)
    g = 0.5 * hsel * (1.0 + jax.lax.erf(hsel * 0.7071067811865476))
    gb = g.astype(jnp.bfloat16)
    gs = jnp.concatenate([masks[e] * gb for e in range(E)]
                         + masks + [zrow], axis=0)
    y = jnp.dot(w2s_ref[...], gs, preferred_element_type=jnp.float32)
    out_ref[...] = y.astype(jnp.bfloat16)

    lo = TB - HALO
    parts = []
    for dw in range(7):
        xw = jnp.concatenate(
            [xp_ref[:, lo + dw:], xc_ref[...], xn_ref[:, :HALO - 6 + dw]],
            axis=1)
        p = None
        for dh in range(7):
            tap = k_ref[:, dh * 7 + dw:dh * 7 + dw + 1]
            t = xw[:, dh * WP:dh * WP + TB] * tap
            p = t if p is None else p + t
        parts.append(p)
    acc = ((parts[0] + parts[1]) + (parts[2] + parts[3])) \
        + ((parts[4] + parts[5]) + parts[6])
    xc = acc.astype(jnp.float32) + cb_ref[...]

    mu = jnp.mean(xc, axis=0, keepdims=True)
    var = jnp.mean(xc * xc, axis=0, keepdims=True) - mu * mu
    xln_s[...] = ((xc - mu) * jax.lax.rsqrt(var + EPS) * lng_ref[...]
                  + lnb_ref[...])


def kernel(input, conv_w, conv_b, ln_g, ln_b, gate_w, w1, b1, w2, b2, ls):
    x3 = jnp.pad(input[0], ((0, 0), (3, HP - H - 3), (3, WP - W - 3)))
    x2 = x3.reshape(DIM, HP * WP).astype(jnp.bfloat16)

    k = conv_w[:, 0, :, :].reshape(DIM, 49)
    k = jnp.pad(k, ((0, 0), (0, 7))).astype(jnp.bfloat16)
    cb = conv_b.reshape(DIM, 1)
    lng = ln_g.reshape(DIM, 1)
    lnb = ln_b.reshape(DIM, 1)
    zcol = jnp.zeros((DIM, E), jnp.float32)
    w1s = jnp.concatenate(
        [jnp.transpose(w1, (1, 0, 2)).reshape(DIM, E * DIM), b1.T, zcol],
        axis=1).astype(jnp.bfloat16)
    w2s = jnp.concatenate(
        [jnp.transpose(w2, (1, 0, 2)).reshape(DIM, E * DIM), b2.T, zcol],
        axis=1).astype(jnp.bfloat16)

    blk = lambda f: pl.BlockSpec((DIM, TB), lambda i: (0, f(i)))
    full = lambda a: pl.BlockSpec(a.shape, lambda i: (0,) * a.ndim)

    y = pl.pallas_call(
        _moecn_kernel,
        grid=(GRID + 1,),
        in_specs=[
            blk(lambda i: jnp.clip(i - 1, 0, GRID - 1)),
            blk(lambda i: jnp.minimum(i, GRID - 1)),
            blk(lambda i: jnp.minimum(i + 1, GRID - 1)),
            full(k), full(cb), full(lng), full(lnb),
            full(gate_w), full(w1s), full(w2s),
        ],
        out_specs=pl.BlockSpec((DIM, TB),
                               lambda i: (0, jnp.maximum(i - 1, 0))),
        out_shape=jax.ShapeDtypeStruct((DIM, HP * WP), jnp.bfloat16),
        scratch_shapes=[pltpu.VMEM((DIM, TB), jnp.float32)],
    )(x2, x2, x2, k, cb, lng, lnb, gate_w, w1s, w2s)

    yc = y.reshape(DIM, HP, WP)[:, 3:3 + H, 3:3 + W]
    return input + ls[None] * yc[None]
